# baseline (speedup 1.0000x reference)
import functools

import numpy as np
import jax
import jax.numpy as jnp
from jax import lax
from jax.experimental import pallas as pl
from jax.experimental.pallas import tpu as pltpu
from jax.experimental.pallas import tpu_sc as plsc

_B = 8
_HH = 224
_W = 224
_D = 96
_DP = 128
_N = _B * _HH * _W
_NT = _B * _HH
_NC = 2
_NS = 16
_NW = _NC * _NS
_CH = 128
_RW = _N // _NW
_NCH_W = _RW // _CH


def _compute_permutation():
    with jax.set_mesh(None), jax.ensure_compile_time_eval():
        p = jax.random.permutation(jax.random.key(42), _N)
        return np.asarray(p, dtype=np.int32)


_PERM = _compute_permutation()


_TB = 16
_NB = _HH // _TB


def _tc_stage(feat_t, mask_t):
    def body(f_ref, m_ref, h_ref):
        f = f_ref[0]
        m = m_ref[0]
        bg = (1.0 - m) * f
        t = jnp.transpose(bg, (0, 2, 1))
        h_ref[:, : _D] = t.reshape(_TB * _W, _D)

    return pl.pallas_call(
        body,
        grid=(_NT // _TB,),
        in_specs=[
            pl.BlockSpec((1, _TB, _D, _W), lambda i: (i // _NB, i % _NB, 0, 0)),
            pl.BlockSpec((1, _TB, 1, _W), lambda i: (i // _NB, i % _NB, 0, 0)),
        ],
        out_specs=pl.BlockSpec((_TB * _W, _DP), lambda i: (i, 0)),
        out_shape=jax.ShapeDtypeStruct((_N, _DP), jnp.float32),
    )(feat_t, mask_t)


_Q = 4
_NQ = _N // _Q
_RWQ = _NQ // _NW
_CHQ = 112
_NCHQ = _RWQ // _CHQ


def _sc_gather_slice(h, perm_q, q):
    mesh = plsc.VectorSubcoreMesh(core_axis_name="c", subcore_axis_name="s")

    @functools.partial(
        pl.kernel,
        out_type=jax.ShapeDtypeStruct((_NQ, _DP), jnp.float32),
        mesh=mesh,
        name=f"sc_gather_q{q}",
        scratch_types=[
            pltpu.VMEM((_RWQ,), jnp.int32),
            pltpu.VMEM((_CHQ, _DP), jnp.float32),
            pltpu.VMEM((_CHQ, _DP), jnp.float32),
            pltpu.SemaphoreType.DMA,
            pltpu.SemaphoreType.DMA,
        ],
    )
    def k(h_hbm, perm_hbm, g_hbm, idx_v, buf0, buf1, sem0, sem1):
        wid = lax.axis_index("c") * _NS + lax.axis_index("s")
        rbase = wid * _RWQ
        pltpu.sync_copy(perm_hbm.at[pl.ds(rbase, _RWQ)], idx_v)

        def body(jj, carry):
            j0 = jj * 2
            idx0 = idx_v.at[pl.ds(j0 * _CHQ, _CHQ)]
            idx1 = idx_v.at[pl.ds((j0 + 1) * _CHQ, _CHQ)]
            cp0 = pltpu.async_copy(h_hbm.at[idx0], buf0, sem0)
            cp1 = pltpu.async_copy(h_hbm.at[idx1], buf1, sem1)
            row0 = rbase + j0 * _CHQ
            cp0.wait()
            pltpu.sync_copy(buf0, g_hbm.at[pl.ds(row0, _CHQ)])
            cp1.wait()
            pltpu.sync_copy(buf1, g_hbm.at[pl.ds(row0 + _CHQ, _CHQ)])
            return carry

        lax.fori_loop(0, _NCHQ // 2, body, 0)

    return k(h, perm_q)


_BQ = _B // _Q


def _tc_combine_slice(feat_tq, mask_tq, g_q):
    def body(f_ref, m_ref, g_ref, o_ref):
        f = f_ref[0]
        m = m_ref[0]
        gr = g_ref[:, : _D].reshape(_TB, _W, _D)
        gt = jnp.transpose(gr, (0, 2, 1))
        o_ref[0] = m * f + (1.0 - m) * gt

    nt_q = _BQ * _HH
    return pl.pallas_call(
        body,
        grid=(nt_q // _TB,),
        in_specs=[
            pl.BlockSpec((1, _TB, _D, _W), lambda i: (i // _NB, i % _NB, 0, 0)),
            pl.BlockSpec((1, _TB, 1, _W), lambda i: (i // _NB, i % _NB, 0, 0)),
            pl.BlockSpec((_TB * _W, _DP), lambda i: (i, 0)),
        ],
        out_specs=pl.BlockSpec((1, _TB, _D, _W), lambda i: (i // _NB, i % _NB, 0, 0)),
        out_shape=jax.ShapeDtypeStruct((_BQ, _HH, _D, _W), jnp.float32),
    )(feat_tq, mask_tq, g_q)


def kernel(feat, mask):
    feat_t = feat.transpose(0, 1, 3, 2)
    mask_t = mask.transpose(0, 1, 3, 2)
    h = _tc_stage(feat_t, mask_t)
    outs = []
    for q in range(_Q):
        perm_q = jnp.asarray(_PERM[q * _NQ:(q + 1) * _NQ])
        g_q = _sc_gather_slice(h, perm_q, q)
        outs.append(_tc_combine_slice(
            feat_t[q * _BQ:(q + 1) * _BQ],
            mask_t[q * _BQ:(q + 1) * _BQ],
            g_q,
        ))
    out_t = jnp.concatenate(outs, axis=0)
    return out_t.transpose(0, 1, 3, 2)


# speedup vs baseline: 1.5854x; 1.5854x over previous
import functools

import numpy as np
import jax
import jax.numpy as jnp
from jax import lax
from jax.experimental import pallas as pl
from jax.experimental.pallas import tpu as pltpu
from jax.experimental.pallas import tpu_sc as plsc

_B = 8
_HH = 224
_W = 224
_D = 96
_DP = 128
_N = _B * _HH * _W
_NT = _B * _HH
_NC = 2
_NS = 16
_NW = _NC * _NS
_CH = 128
_RW = _N // _NW
_NCH_W = _RW // _CH


def _compute_permutation():
    with jax.set_mesh(None), jax.ensure_compile_time_eval():
        p = jax.random.permutation(jax.random.key(42), _N)
        return np.asarray(p, dtype=np.int32)


_PERM = _compute_permutation()


_TB = 32
_NB = _HH // _TB


def _tc_stage(feat_t, mask_t):
    def body(f_ref, m_ref, h_ref):
        f = f_ref[0]
        m = m_ref[0]
        bg = (1.0 - m) * f
        t = jnp.transpose(bg, (0, 2, 1))
        h_ref[:, : _D] = t.reshape(_TB * _W, _D)

    return pl.pallas_call(
        body,
        grid=(_NT // _TB,),
        in_specs=[
            pl.BlockSpec((1, _TB, _D, _W), lambda i: (i // _NB, i % _NB, 0, 0)),
            pl.BlockSpec((1, _TB, 1, _W), lambda i: (i // _NB, i % _NB, 0, 0)),
        ],
        out_specs=pl.BlockSpec((_TB * _W, _DP), lambda i: (i, 0)),
        out_shape=jax.ShapeDtypeStruct((_N, _DP), jnp.float32),
    )(feat_t, mask_t)


def _sc_gather(h, perm):
    mesh = plsc.VectorSubcoreMesh(core_axis_name="c", subcore_axis_name="s")

    @functools.partial(
        pl.kernel,
        out_type=jax.ShapeDtypeStruct((_N, _DP), jnp.float32),
        mesh=mesh,
        scratch_types=[
            pltpu.VMEM((_RW,), jnp.int32),
            pltpu.VMEM((_CH, _DP), jnp.float32),
            pltpu.VMEM((_CH, _DP), jnp.float32),
            pltpu.VMEM((_CH, _DP), jnp.float32),
            pltpu.VMEM((_CH, _DP), jnp.float32),
            pltpu.SemaphoreType.DMA,
            pltpu.SemaphoreType.DMA,
            pltpu.SemaphoreType.DMA,
            pltpu.SemaphoreType.DMA,
            pltpu.SemaphoreType.DMA,
            pltpu.SemaphoreType.DMA,
            pltpu.SemaphoreType.DMA,
            pltpu.SemaphoreType.DMA,
        ],
    )
    def k(h_hbm, perm_hbm, g_hbm, idx_v,
          buf0, buf1, buf2, buf3,
          gsem0, gsem1, gsem2, gsem3,
          ssem0, ssem1, ssem2, ssem3):
        wid = lax.axis_index("c") * _NS + lax.axis_index("s")
        rbase = wid * _RW
        pltpu.sync_copy(perm_hbm.at[pl.ds(rbase, _RW)], idx_v)
        bufs = (buf0, buf1, buf2, buf3)
        gsems = (gsem0, gsem1, gsem2, gsem3)
        ssems = (ssem0, ssem1, ssem2, ssem3)

        def start_gather(j, s):
            idx = idx_v.at[pl.ds(j * _CH, _CH)]
            pltpu.async_copy(h_hbm.at[idx], bufs[s], gsems[s])

        for s in range(4):
            start_gather(s, s)

        def body(gg, carry):
            j0 = gg * 4
            for s in range(4):
                j = j0 + s
                pltpu.make_async_copy(h_hbm.at[pl.ds(0, _CH)], bufs[s],
                                      gsems[s]).wait()
                @pl.when(gg > 0)
                def _(s=s):
                    pltpu.make_async_copy(bufs[s],
                                          g_hbm.at[pl.ds(0, _CH)],
                                          ssems[s]).wait()
                pltpu.async_copy(bufs[s],
                                 g_hbm.at[pl.ds(rbase + j * _CH, _CH)],
                                 ssems[s])

            @pl.when(j0 + 4 < _NCH_W)
            def _():
                for s in range(4):
                    jn = j0 + 4 + s

                    @pl.when(jn < _NCH_W)
                    def _(jn=jn, s=s):
                        idx = idx_v.at[pl.ds(jn * _CH, _CH)]
                        pltpu.async_copy(h_hbm.at[idx], bufs[s], gsems[s])
            return carry

        ngroups = (_NCH_W + 3) // 4
        lax.fori_loop(0, ngroups, body, 0)
        for s in range(4):
            jlast = (ngroups - 1) * 4 + s

            @pl.when(jlast < _NCH_W)
            def _(s=s):
                pltpu.make_async_copy(bufs[s], g_hbm.at[pl.ds(0, _CH)],
                                      ssems[s]).wait()

    return k(h, perm)


def _tc_combine(feat_t, mask_t, g):
    def body(f_ref, m_ref, g_ref, o_ref):
        f = f_ref[0]
        m = m_ref[0]
        gr = g_ref[:, : _D].reshape(_TB, _W, _D)
        gt = jnp.transpose(gr, (0, 2, 1))
        o_ref[0] = m * f + (1.0 - m) * gt

    return pl.pallas_call(
        body,
        grid=(_NT // _TB,),
        in_specs=[
            pl.BlockSpec((1, _TB, _D, _W), lambda i: (i // _NB, i % _NB, 0, 0)),
            pl.BlockSpec((1, _TB, 1, _W), lambda i: (i // _NB, i % _NB, 0, 0)),
            pl.BlockSpec((_TB * _W, _DP), lambda i: (i, 0)),
        ],
        out_specs=pl.BlockSpec((1, _TB, _D, _W), lambda i: (i // _NB, i % _NB, 0, 0)),
        out_shape=jax.ShapeDtypeStruct((_B, _HH, _D, _W), jnp.float32),
    )(feat_t, mask_t, g)


def kernel(feat, mask):
    feat_t = feat.transpose(0, 1, 3, 2)
    mask_t = mask.transpose(0, 1, 3, 2)
    perm = jnp.asarray(_PERM)
    h = _tc_stage(feat_t, mask_t)
    g = _sc_gather(h, perm)
    out_t = _tc_combine(feat_t, mask_t, g)
    return out_t.transpose(0, 1, 3, 2)
